# Initial kernel scaffold; baseline (speedup 1.0000x reference)
import functools

import jax
import jax.numpy as jnp
import numpy as np
from jax import lax
from jax.experimental import pallas as pl
from jax.experimental.pallas import tpu as pltpu
from jax.experimental.pallas import tpu_sc as plsc

_PATCH_PERCENTAGE = 0.25
_GAUSSIAN_STD = 0.25
_NUM_WORKERS = 32


def _sampling_logits(scores, centers):
    B, N = scores.shape
    n = int(np.sqrt(N))
    y_patch = jnp.linspace(0.0, 1.0, n)
    x_patch = jnp.linspace(0.0, 1.0, n)
    grid_y, grid_x = jnp.meshgrid(y_patch, x_patch, indexing='ij')
    grid_coords = jnp.stack([grid_y.flatten(), grid_x.flatten()], axis=1)
    distances_sq = ((grid_coords[None, :, :] - centers[:, None, :]) ** 2).sum(axis=2)
    gaussian = jnp.exp(-distances_sq / (2.0 * _GAUSSIAN_STD ** 2))
    weights = gaussian / (gaussian.sum(axis=1, keepdims=True) + 1e-08)
    joint = scores * weights
    joint = joint / (joint.sum(axis=1, keepdims=True) + 1e-08)
    gumbel = jax.random.gumbel(jax.random.key(42), joint.shape, dtype=joint.dtype)
    return jnp.log(joint + 1e-20) + gumbel


def _topk_body(logits_ref, out_ref):
    b = pl.program_id(0)
    v = logits_ref[...]
    N = v.shape[1]
    K = out_ref.shape[2]
    rowmat = jnp.broadcast_to(v, (N, N))
    ii = lax.broadcasted_iota(jnp.int32, (N, N), 0)
    jj = lax.broadcasted_iota(jnp.int32, (N, N), 1)
    vcol = jnp.sum(jnp.where(ii == jj, rowmat, 0.0), axis=1, keepdims=True)
    colmat = jnp.broadcast_to(vcol, (N, N))
    beats = (rowmat > colmat) | ((rowmat == colmat) & (jj < ii))
    rank = jnp.sum(beats.astype(jnp.int32), axis=1, keepdims=True)
    rmat = jnp.broadcast_to(rank, (N, K))
    ss = lax.broadcasted_iota(jnp.int32, (N, K), 1)
    ival = lax.broadcasted_iota(jnp.int32, (N, K), 0)
    idx = jnp.sum(jnp.where(rmat == ss, ival, 0), axis=0, keepdims=True)
    out_ref[...] = (idx + b * N)[None]


def _topk_global_indices(logits, K):
    B, N = logits.shape
    return pl.pallas_call(
        _topk_body,
        grid=(B,),
        in_specs=[pl.BlockSpec((1, N), lambda b: (b, 0))],
        out_specs=pl.BlockSpec((1, 1, K), lambda b: (b, 0, 0)),
        out_shape=jax.ShapeDtypeStruct((B, 1, K), jnp.int32),
    )(logits)


def _sc_gather_add(patches_flat, pos_table, gidx):
    R = gidx.shape[0]
    D = patches_flat.shape[1]
    N = pos_table.shape[0]
    rows_w = R // _NUM_WORKERS
    C = 16
    n_chunks = rows_w // C
    mesh = plsc.VectorSubcoreMesh(core_axis_name="c", subcore_axis_name="s")

    @functools.partial(
        pl.kernel,
        mesh=mesh,
        out_type=jax.ShapeDtypeStruct((R, D), jnp.float32),
        scratch_types=[
            pltpu.VMEM((rows_w,), jnp.int32),
            pltpu.VMEM((C, D), jnp.float32),
            pltpu.VMEM((C, D), jnp.float32),
            pltpu.SemaphoreType.DMA,
            pltpu.SemaphoreType.DMA,
        ],
    )
    def k(pf_hbm, pos_hbm, gidx_hbm, out_hbm, idx_v, pbuf, qbuf, s1, s2):
        wid = lax.axis_index("s") * 2 + lax.axis_index("c")
        base = wid * rows_w
        pltpu.sync_copy(gidx_hbm.at[pl.ds(base, rows_w)], idx_v)

        def chunk(ci, carry):
            off = ci * C
            iv = idx_v[pl.ds(off, C)]
            lv = jnp.bitwise_and(iv, N - 1)
            c1 = pltpu.async_copy(pf_hbm.at[iv], pbuf, s1)
            c2 = pltpu.async_copy(pos_hbm.at[lv], qbuf, s2)
            c1.wait()
            c2.wait()

            def addrow(r, c):
                for dd in range(D // 16):
                    sl = pl.ds(dd * 16, 16)
                    pbuf[r, sl] = pbuf[r, sl] + qbuf[r, sl]
                return c

            lax.fori_loop(0, C, addrow, 0)
            pltpu.sync_copy(pbuf, out_hbm.at[pl.ds(base + off, C)])
            return carry

        lax.fori_loop(0, n_chunks, chunk, 0)

    return k(patches_flat, pos_table, gidx)


def kernel(color_patches, vit_positional_embedding, scores, centers):
    B, N, D = color_patches.shape
    K = max(1, int(N * _PATCH_PERCENTAGE))
    logits = _sampling_logits(scores, centers)
    gidx = _topk_global_indices(logits, K).reshape(B * K)
    patches_flat = color_patches.reshape(B * N, D)
    pos_table = vit_positional_embedding[0, 1:, :]
    out = _sc_gather_add(patches_flat, pos_table, gidx)
    return out.reshape(B, K, D)


# speedup vs baseline: 1.2951x; 1.2951x over previous
import functools

import jax
import jax.numpy as jnp
import numpy as np
from jax import lax
from jax.experimental import pallas as pl
from jax.experimental.pallas import tpu as pltpu
from jax.experimental.pallas import tpu_sc as plsc

_PATCH_PERCENTAGE = 0.25
_GAUSSIAN_STD = 0.25
_NUM_WORKERS = 32


def _sampling_logits(scores, centers):
    B, N = scores.shape
    n = int(np.sqrt(N))
    y_patch = jnp.linspace(0.0, 1.0, n)
    x_patch = jnp.linspace(0.0, 1.0, n)
    grid_y, grid_x = jnp.meshgrid(y_patch, x_patch, indexing='ij')
    grid_coords = jnp.stack([grid_y.flatten(), grid_x.flatten()], axis=1)
    distances_sq = ((grid_coords[None, :, :] - centers[:, None, :]) ** 2).sum(axis=2)
    gaussian = jnp.exp(-distances_sq / (2.0 * _GAUSSIAN_STD ** 2))
    weights = gaussian / (gaussian.sum(axis=1, keepdims=True) + 1e-08)
    joint = scores * weights
    joint = joint / (joint.sum(axis=1, keepdims=True) + 1e-08)
    gumbel = jax.random.gumbel(jax.random.key(42), joint.shape, dtype=joint.dtype)
    return jnp.log(joint + 1e-20) + gumbel


def _topk_body(logits_ref, out_ref):
    b = pl.program_id(0)
    v = logits_ref[0]
    N = v.shape[1]
    K = out_ref.shape[2]
    rowmat = jnp.broadcast_to(v, (N, N))
    ii = lax.broadcasted_iota(jnp.int32, (N, N), 0)
    jj = lax.broadcasted_iota(jnp.int32, (N, N), 1)
    vcol = jnp.sum(jnp.where(ii == jj, rowmat, 0.0), axis=1, keepdims=True)
    colmat = jnp.broadcast_to(vcol, (N, N))
    beats = (rowmat > colmat) | ((rowmat == colmat) & (jj < ii))
    rank = jnp.sum(beats.astype(jnp.int32), axis=1, keepdims=True)
    rmat = jnp.broadcast_to(rank, (N, K))
    ss = lax.broadcasted_iota(jnp.int32, (N, K), 1)
    ival = lax.broadcasted_iota(jnp.int32, (N, K), 0)
    idx = jnp.sum(jnp.where(rmat == ss, ival, 0), axis=0, keepdims=True)
    out_ref[...] = (idx + b * N)[None]


def _topk_global_indices(logits, K):
    B, N = logits.shape
    return pl.pallas_call(
        _topk_body,
        grid=(B,),
        in_specs=[pl.BlockSpec((1, 1, N), lambda b: (b, 0, 0))],
        out_specs=pl.BlockSpec((1, 1, K), lambda b: (b, 0, 0)),
        out_shape=jax.ShapeDtypeStruct((B, 1, K), jnp.int32),
    )(logits.reshape(B, 1, N))


def _sc_gather_add(patches_flat, pos_table, gidx):
    R = gidx.shape[0]
    D = patches_flat.shape[1]
    N = pos_table.shape[0]
    rows_w = R // _NUM_WORKERS
    C = 16
    n_chunks = rows_w // C
    mesh = plsc.VectorSubcoreMesh(core_axis_name="c", subcore_axis_name="s")

    @functools.partial(
        pl.kernel,
        mesh=mesh,
        out_type=jax.ShapeDtypeStruct((R, D), jnp.float32),
        scratch_types=[
            pltpu.VMEM((rows_w,), jnp.int32),
            pltpu.VMEM((C, D), jnp.float32),
            pltpu.VMEM((C, D), jnp.float32),
            pltpu.SemaphoreType.DMA,
            pltpu.SemaphoreType.DMA,
        ],
    )
    def k(pf_hbm, pos_hbm, gidx_hbm, out_hbm, idx_v, pbuf, qbuf, s1, s2):
        wid = lax.axis_index("s") * 2 + lax.axis_index("c")
        base = wid * rows_w
        pltpu.sync_copy(gidx_hbm.at[pl.ds(base, rows_w)], idx_v)

        def chunk(ci, carry):
            off = ci * C
            iv = idx_v[pl.ds(off, C)]
            lv = jnp.bitwise_and(iv, N - 1)
            c1 = pltpu.async_copy(pf_hbm.at[iv], pbuf, s1)
            c2 = pltpu.async_copy(pos_hbm.at[lv], qbuf, s2)
            c1.wait()
            c2.wait()

            def addrow(r, c):
                for dd in range(D // 16):
                    sl = pl.ds(dd * 16, 16)
                    pbuf[r, sl] = pbuf[r, sl] + qbuf[r, sl]
                return c

            lax.fori_loop(0, C, addrow, 0)
            pltpu.sync_copy(pbuf, out_hbm.at[pl.ds(base + off, C)])
            return carry

        lax.fori_loop(0, n_chunks, chunk, 0)

    return k(patches_flat, pos_table, gidx)


def kernel(color_patches, vit_positional_embedding, scores, centers):
    B, N, D = color_patches.shape
    K = max(1, int(N * _PATCH_PERCENTAGE))
    logits = _sampling_logits(scores, centers)
    gidx = _topk_global_indices(logits, K).reshape(B * K)
    patches_flat = color_patches.reshape(B * N, D)
    pos_table = vit_positional_embedding[0, 1:, :]
    out = _sc_gather_add(patches_flat, pos_table, gidx)
    return out.reshape(B, K, D)
